# Initial kernel scaffold; baseline (speedup 1.0000x reference)
import jax
import jax.numpy as jnp
from jax.experimental import pallas as pl


def kernel(x):
    raise NotImplementedError("write your pallas kernel here")


# speedup vs baseline: 16.7342x; 16.7342x over previous
import jax
import jax.numpy as jnp
from jax.experimental import pallas as pl
from jax.experimental.pallas import tpu as pltpu

_K = 327
_ROWS = 128
_COLS = 32768
_BLOCK_R = 16


def _topk_mask_kernel(x_ref, o_ref):
    x = x_ref[...]
    i = jax.lax.bitcast_convert_type(x, jnp.int32)
    ikeys = jnp.where(i < 0, i ^ jnp.int32(0x7FFFFFFF), i)

    def body(j, tb):
        bit = jax.lax.shift_left(jnp.int32(1), jnp.int32(31) - j)
        t2b = tb | bit
        t2s = t2b ^ jnp.int32(-2147483648)
        cnt = jnp.sum((ikeys >= t2s).astype(jnp.int32), axis=1, keepdims=True)
        return jnp.where(cnt >= _K, t2b, tb)

    tb0 = jnp.zeros((x.shape[0], 1), jnp.int32)
    tb = jax.lax.fori_loop(0, 32, body, tb0)
    ts = tb ^ jnp.int32(-2147483648)
    tbits = jnp.where(ts < 0, ts ^ jnp.int32(0x7FFFFFFF), ts)
    t = jax.lax.bitcast_convert_type(tbits, jnp.float32)
    o_ref[...] = jnp.where(x >= t, x, jnp.float32(0.0))


def kernel(x):
    return pl.pallas_call(
        _topk_mask_kernel,
        grid=(_ROWS // _BLOCK_R,),
        in_specs=[pl.BlockSpec((_BLOCK_R, _COLS), lambda r: (r, 0))],
        out_specs=pl.BlockSpec((_BLOCK_R, _COLS), lambda r: (r, 0)),
        out_shape=jax.ShapeDtypeStruct((_ROWS, _COLS), jnp.float32),
    )(x)


# speedup vs baseline: 16.7370x; 1.0002x over previous
import jax
import jax.numpy as jnp
from jax.experimental import pallas as pl
from jax.experimental.pallas import tpu as pltpu

_K = 327
_ROWS = 128
_COLS = 32768
_BLOCK_R = 16


def _topk_mask_kernel(x_ref, o_ref):
    x = x_ref[...]
    i = jax.lax.bitcast_convert_type(x, jnp.int32)
    ikeys = jnp.where(i < 0, i ^ jnp.int32(0x7FFFFFFF), i)

    def body(j, tb):
        bit = jax.lax.shift_left(jnp.int32(1), jnp.int32(31) - j)
        t2b = tb | bit
        t2s = t2b ^ jnp.int32(-2147483648)
        cnt = jnp.sum((ikeys >= t2s).astype(jnp.int32), axis=1, keepdims=True)
        return jnp.where(cnt >= _K, t2b, tb)

    tb0 = jnp.zeros((x.shape[0], 1), jnp.int32)
    tb = jax.lax.fori_loop(0, 32, body, tb0)
    ts = tb ^ jnp.int32(-2147483648)
    tbits = jnp.where(ts < 0, ts ^ jnp.int32(0x7FFFFFFF), ts)
    t = jax.lax.bitcast_convert_type(tbits, jnp.float32)
    o_ref[...] = jnp.where(x >= t, x, jnp.float32(0.0))


def kernel(x):
    return pl.pallas_call(
        _topk_mask_kernel,
        grid=(_ROWS // _BLOCK_R,),
        in_specs=[pl.BlockSpec((_BLOCK_R, _COLS), lambda r: (r, 0))],
        out_specs=pl.BlockSpec((_BLOCK_R, _COLS), lambda r: (r, 0)),
        out_shape=jax.ShapeDtypeStruct((_ROWS, _COLS), jnp.float32),
        compiler_params=pltpu.CompilerParams(
            dimension_semantics=("parallel",),
        ),
    )(x)


# speedup vs baseline: 18.3082x; 1.0939x over previous
import jax
import jax.numpy as jnp
from jax.experimental import pallas as pl
from jax.experimental.pallas import tpu as pltpu

_K = 327
_ROWS = 128
_COLS = 32768
_BLOCK_R = 16


def _topk_mask_kernel(x_ref, o_ref):
    x = x_ref[...]
    i = jax.lax.bitcast_convert_type(x, jnp.int32)
    ikeys = jnp.where(i < 0, i ^ jnp.int32(0x7FFFFFFF), i)

    nrows = x.shape[0]
    imin = jnp.int32(-2147483648)

    def cond(state):
        j, _, done, _ = state
        return jnp.logical_and(j < 32, jnp.sum(done) < nrows)

    def body(state):
        j, tb, done, tfin = state
        bit = jax.lax.shift_left(jnp.int32(1), jnp.int32(31) - j)
        t2b = tb | bit
        t2s = t2b ^ imin
        cnt = jnp.sum((ikeys >= t2s).astype(jnp.int32), axis=1, keepdims=True)
        newly = jnp.where(cnt == _K, 1 - done, 0)
        tfin = jnp.where(newly == 1, t2s, tfin)
        done = done | newly
        tb = jnp.where(cnt >= _K, t2b, tb)
        return j + jnp.int32(1), tb, done, tfin

    state0 = (
        jnp.int32(0),
        jnp.zeros((nrows, 1), jnp.int32),
        jnp.zeros((nrows, 1), jnp.int32),
        jnp.zeros((nrows, 1), jnp.int32),
    )
    _, tb, done, tfin = jax.lax.while_loop(cond, body, state0)
    ts = tb ^ imin
    tfin = jnp.where(done == 1, tfin, ts)
    thr_key = jnp.min(
        jnp.where(ikeys >= tfin, ikeys, jnp.int32(2147483647)),
        axis=1,
        keepdims=True,
    )
    tbits = jnp.where(thr_key < 0, thr_key ^ jnp.int32(0x7FFFFFFF), thr_key)
    t = jax.lax.bitcast_convert_type(tbits, jnp.float32)
    o_ref[...] = jnp.where(x >= t, x, jnp.float32(0.0))


def kernel(x):
    return pl.pallas_call(
        _topk_mask_kernel,
        grid=(_ROWS // _BLOCK_R,),
        in_specs=[pl.BlockSpec((_BLOCK_R, _COLS), lambda r: (r, 0))],
        out_specs=pl.BlockSpec((_BLOCK_R, _COLS), lambda r: (r, 0)),
        out_shape=jax.ShapeDtypeStruct((_ROWS, _COLS), jnp.float32),
    )(x)


# speedup vs baseline: 21.8687x; 1.1945x over previous
import jax
import jax.numpy as jnp
from jax.experimental import pallas as pl
from jax.experimental.pallas import tpu as pltpu

_K = 327
_ROWS = 128
_COLS = 32768
_BLOCK_R = 16
_INTERP_ITERS = 6
_MAX_ITERS = 48


def _f2k(v):
    b = jax.lax.bitcast_convert_type(v, jnp.int32)
    return jnp.where(b < 0, b ^ jnp.int32(0x7FFFFFFF), b)


def _k2f(k):
    b = jnp.where(k < 0, k ^ jnp.int32(0x7FFFFFFF), k)
    return jax.lax.bitcast_convert_type(b, jnp.float32)


def _topk_mask_kernel(x_ref, o_ref):
    x = x_ref[...]
    nrows = x.shape[0]
    ncols = x.shape[1]
    one = jnp.int32(1)

    rmin = jnp.min(x, axis=1, keepdims=True)
    rmax = jnp.max(x, axis=1, keepdims=True)
    s1 = jnp.sum(x, axis=1, keepdims=True)
    s2 = jnp.sum(x * x, axis=1, keepdims=True)
    mu = s1 / ncols
    var = jnp.maximum(s2 / ncols - mu * mu, 0.0)
    sd = jnp.sqrt(var)
    a_v = mu + 1.8 * sd
    b_v = mu + 3.6 * sd
    a_v = jnp.where(jnp.isfinite(a_v), jnp.clip(a_v, rmin, rmax), rmax)
    b_v = jnp.where(jnp.isfinite(b_v), jnp.clip(b_v, rmin, rmax), rmax)

    cnt_a = jnp.sum((x >= a_v).astype(jnp.int32), axis=1, keepdims=True)
    cnt_b = jnp.sum((x >= b_v).astype(jnp.int32), axis=1, keepdims=True)

    lo_ok = cnt_a >= _K
    lo_k = jnp.where(lo_ok, _f2k(a_v), _f2k(rmin))
    cnt_lo = jnp.where(lo_ok, cnt_a, jnp.int32(ncols))
    hi_ok = cnt_b < _K
    hi_k = jnp.where(hi_ok, _f2k(b_v), _f2k(rmax) + one)
    cnt_hi = jnp.where(hi_ok, cnt_b, jnp.int32(0))

    logk = jnp.float32(jnp.log(float(_K)))

    def cond(state):
        j, _, _, _, _, done, _, _ = state
        return jnp.logical_and(j < _MAX_ITERS, jnp.sum(done) < nrows)

    def body(state):
        j, lo_k, hi_k, cnt_lo, cnt_hi, done, mode_min, bound = state
        lo_v = _k2f(lo_k)
        hi_v = _k2f(hi_k)

        width1 = jnp.where(hi_k == lo_k + one, 1 - done, 0)
        bound = jnp.where(width1 == 1, lo_v, bound)
        mode_min = mode_min | width1
        done = done | width1

        ch = jnp.maximum(cnt_hi.astype(jnp.float32), 0.5)
        cl = cnt_lo.astype(jnp.float32)
        r = (logk - jnp.log(ch)) / (jnp.log(cl) - jnp.log(ch))
        p_interp = hi_v + (lo_v - hi_v) * r
        pk_i = _f2k(p_interp)
        pk_m = (lo_k >> 1) + (hi_k >> 1) + (lo_k & hi_k & one)
        pk = jnp.where(j < _INTERP_ITERS, pk_i, pk_m)
        pk = jnp.clip(pk, lo_k + one, hi_k - one)
        p_v = _k2f(pk)

        cnt = jnp.sum((x >= p_v).astype(jnp.int32), axis=1, keepdims=True)

        hit_k = jnp.where(cnt == _K, 1 - done, 0)
        bound = jnp.where(hit_k == 1, p_v, bound)
        mode_min = mode_min | hit_k
        done = done | hit_k
        hit_k1 = jnp.where(cnt == _K - 1, 1 - done, 0)
        bound = jnp.where(hit_k1 == 1, p_v, bound)
        done = done | hit_k1

        live = done == 0
        take_lo = jnp.logical_and(live, cnt >= _K)
        take_hi = jnp.logical_and(live, cnt < _K)
        lo_k = jnp.where(take_lo, pk, lo_k)
        cnt_lo = jnp.where(take_lo, cnt, cnt_lo)
        hi_k = jnp.where(take_hi, pk, hi_k)
        cnt_hi = jnp.where(take_hi, cnt, cnt_hi)
        return j + one, lo_k, hi_k, cnt_lo, cnt_hi, done, mode_min, bound

    state0 = (
        jnp.int32(0),
        lo_k,
        hi_k,
        cnt_lo,
        cnt_hi,
        jnp.zeros((nrows, 1), jnp.int32),
        jnp.zeros((nrows, 1), jnp.int32),
        jnp.zeros((nrows, 1), jnp.float32),
    )
    st = jax.lax.while_loop(cond, body, state0)
    _, lo_k, _, _, _, done, mode_min, bound = st
    bound = jnp.where(done == 0, _k2f(lo_k), bound)
    mode_min = jnp.where(done == 0, one, mode_min)

    s = jnp.where(mode_min == 1, jnp.float32(1.0), jnp.float32(-1.0))
    z = x * s
    zb = bound * s
    ok = jnp.logical_or(
        z > zb, jnp.logical_and(mode_min == 1, z == zb)
    )
    m = jnp.min(jnp.where(ok, z, jnp.float32(jnp.inf)), axis=1, keepdims=True)
    thr = m * s

    o_ref[...] = jnp.where(x >= thr, x, jnp.float32(0.0))


def kernel(x):
    return pl.pallas_call(
        _topk_mask_kernel,
        grid=(_ROWS // _BLOCK_R,),
        in_specs=[pl.BlockSpec((_BLOCK_R, _COLS), lambda r: (r, 0))],
        out_specs=pl.BlockSpec((_BLOCK_R, _COLS), lambda r: (r, 0)),
        out_shape=jax.ShapeDtypeStruct((_ROWS, _COLS), jnp.float32),
    )(x)


# speedup vs baseline: 23.1396x; 1.0581x over previous
import jax
import jax.numpy as jnp
from jax.experimental import pallas as pl
from jax.experimental.pallas import tpu as pltpu

_K = 327
_ROWS = 128
_COLS = 32768
_BLOCK_R = 16
_STAT_COLS = 4096
_INTERP_ITERS = 6
_MAX_ITERS = 50


def _f2k(v):
    b = jax.lax.bitcast_convert_type(v, jnp.int32)
    return jnp.where(b < 0, b ^ jnp.int32(0x7FFFFFFF), b)


def _k2f(k):
    b = jnp.where(k < 0, k ^ jnp.int32(0x7FFFFFFF), k)
    return jax.lax.bitcast_convert_type(b, jnp.float32)


def _topk_mask_kernel(x_ref, o_ref):
    x = x_ref[...]
    nrows = x.shape[0]
    ncols = x.shape[1]
    one = jnp.int32(1)
    kf = jnp.float32(_K)

    xs = x[:, :_STAT_COLS]
    s1 = jnp.sum(xs, axis=1, keepdims=True)
    s2 = jnp.sum(xs * xs, axis=1, keepdims=True)
    mu = s1 / _STAT_COLS
    sd = jnp.sqrt(jnp.maximum(s2 / _STAT_COLS - mu * mu, 0.0))
    a_v = mu + 1.7 * sd
    b_v = mu + 3.7 * sd

    cnt_a = jnp.sum(
        jnp.where(x >= a_v, 1.0, 0.0).astype(jnp.float32), axis=1, keepdims=True
    )
    cnt_b = jnp.sum(
        jnp.where(x >= b_v, 1.0, 0.0).astype(jnp.float32), axis=1, keepdims=True
    )

    neg_inf_k = _f2k(jnp.float32(-jnp.inf))
    pos_nan_k = _f2k(jnp.float32(jnp.inf)) + one
    lo_ok = jnp.logical_and(cnt_a >= kf, a_v == a_v)
    lo_k = jnp.where(lo_ok, _f2k(a_v), neg_inf_k)
    cnt_lo = jnp.where(lo_ok, cnt_a, jnp.float32(ncols))
    hi_ok = jnp.logical_and(cnt_b < kf, b_v == b_v)
    hi_k = jnp.where(hi_ok, _f2k(b_v), pos_nan_k)
    cnt_hi = jnp.where(hi_ok, cnt_b, jnp.float32(0.0))

    logk = jnp.float32(jnp.log(float(_K)))
    llo = jnp.log(cnt_lo)
    lhi = jnp.log(jnp.maximum(cnt_hi, 0.5))

    def cond(state):
        j = state[0]
        done = state[5]
        return jnp.logical_and(j < _MAX_ITERS, jnp.sum(done) < nrows)

    def body(state):
        j, lo_k, hi_k, llo, lhi, done, mode_min, bound = state
        lo_v = _k2f(lo_k)
        hi_v = _k2f(hi_k)

        width1 = jnp.where(hi_k == lo_k + one, 1 - done, 0)
        bound = jnp.where(width1 == 1, lo_v, bound)
        mode_min = mode_min | width1
        done = done | width1

        r = (logk - lhi) / (llo - lhi)
        p_interp = hi_v + (lo_v - hi_v) * r
        pk_i = _f2k(p_interp)
        pk_m = (lo_k >> 1) + (hi_k >> 1) + (lo_k & hi_k & one)
        pk = jnp.where(j < _INTERP_ITERS, pk_i, pk_m)
        pk = jnp.clip(pk, lo_k + one, hi_k - one)
        p_v = _k2f(pk)

        cnt = jnp.sum(
            jnp.where(x >= p_v, 1.0, 0.0).astype(jnp.float32),
            axis=1,
            keepdims=True,
        )

        hit_k = jnp.where(cnt == kf, 1 - done, 0)
        bound = jnp.where(hit_k == 1, p_v, bound)
        mode_min = mode_min | hit_k
        done = done | hit_k
        hit_k1 = jnp.where(cnt == kf - 1.0, 1 - done, 0)
        bound = jnp.where(hit_k1 == 1, p_v, bound)
        done = done | hit_k1

        lp = jnp.log(jnp.maximum(cnt, 0.5))
        live = done == 0
        take_lo = jnp.logical_and(live, cnt >= kf)
        take_hi = jnp.logical_and(live, cnt < kf)
        lo_k = jnp.where(take_lo, pk, lo_k)
        llo = jnp.where(take_lo, lp, llo)
        hi_k = jnp.where(take_hi, pk, hi_k)
        lhi = jnp.where(take_hi, lp, lhi)
        return j + one, lo_k, hi_k, llo, lhi, done, mode_min, bound

    state0 = (
        jnp.int32(0),
        lo_k,
        hi_k,
        llo,
        lhi,
        jnp.zeros((nrows, 1), jnp.int32),
        jnp.zeros((nrows, 1), jnp.int32),
        jnp.zeros((nrows, 1), jnp.float32),
    )
    st = jax.lax.while_loop(cond, body, state0)
    _, lo_k, _, _, _, done, mode_min, bound = st
    bound = jnp.where(done == 0, _k2f(lo_k), bound)
    mode_min = jnp.where(done == 0, one, mode_min)

    s = jnp.where(mode_min == 1, jnp.float32(1.0), jnp.float32(-1.0))
    z = x * s
    zb = bound * s
    ok = jnp.logical_or(
        z > zb, jnp.logical_and(mode_min == 1, z == zb)
    )
    m = jnp.min(jnp.where(ok, z, jnp.float32(jnp.inf)), axis=1, keepdims=True)
    thr = m * s

    o_ref[...] = jnp.where(x >= thr, x, jnp.float32(0.0))


def kernel(x):
    return pl.pallas_call(
        _topk_mask_kernel,
        grid=(_ROWS // _BLOCK_R,),
        in_specs=[pl.BlockSpec((_BLOCK_R, _COLS), lambda r: (r, 0))],
        out_specs=pl.BlockSpec((_BLOCK_R, _COLS), lambda r: (r, 0)),
        out_shape=jax.ShapeDtypeStruct((_ROWS, _COLS), jnp.float32),
    )(x)


# speedup vs baseline: 27.1055x; 1.1714x over previous
import jax
import jax.numpy as jnp
from jax.experimental import pallas as pl
from jax.experimental.pallas import tpu as pltpu

_K = 327
_ROWS = 128
_COLS = 32768
_BLOCK_R = 16
_STAT_COLS = 2048
_INTERP_ITERS = 8
_MAX_ITERS = 50


def _f2k(v):
    b = jax.lax.bitcast_convert_type(v, jnp.int32)
    return jnp.where(b < 0, b ^ jnp.int32(0x7FFFFFFF), b)


def _k2f(k):
    b = jnp.where(k < 0, k ^ jnp.int32(0x7FFFFFFF), k)
    return jax.lax.bitcast_convert_type(b, jnp.float32)


def _topk_mask_kernel(x_ref, o_ref):
    x = x_ref[...]
    nrows = x.shape[0]
    ncols = x.shape[1]
    one = jnp.int32(1)
    kf = jnp.float32(_K)

    xs = x[:, :_STAT_COLS]
    s1 = jnp.sum(xs, axis=1, keepdims=True)
    s2 = jnp.sum(xs * xs, axis=1, keepdims=True)
    mu = s1 / _STAT_COLS
    sd = jnp.sqrt(jnp.maximum(s2 / _STAT_COLS - mu * mu, 0.0))
    a_v = mu + 1.65 * sd
    b_v = mu + 3.8 * sd

    cnt_a = jnp.sum(jnp.where(x >= a_v, 1.0, 0.0), axis=1, keepdims=True)
    cnt_b = jnp.sum(jnp.where(x >= b_v, 1.0, 0.0), axis=1, keepdims=True)

    neg_inf_k = _f2k(jnp.float32(-jnp.inf))
    pos_nan_k = _f2k(jnp.float32(jnp.inf)) + one
    lo_ok = jnp.logical_and(cnt_a >= kf, a_v == a_v)
    lo_k = jnp.where(lo_ok, _f2k(a_v), neg_inf_k)
    cnt_lo = jnp.where(lo_ok, cnt_a, jnp.float32(ncols))
    hi_ok = jnp.logical_and(cnt_b < kf, b_v == b_v)
    hi_k = jnp.where(hi_ok, _f2k(b_v), pos_nan_k)
    cnt_hi = jnp.where(hi_ok, cnt_b, jnp.float32(0.0))

    logk = jnp.float32(jnp.log(float(_K)))
    llo = jnp.log(cnt_lo)
    lhi = jnp.log(jnp.maximum(cnt_hi, 0.5))

    def cond(state):
        j = state[0]
        done = state[5]
        return jnp.logical_and(j < _MAX_ITERS, jnp.sum(done) < nrows)

    def body(state):
        j, lo_k, hi_k, llo, lhi, done, bound = state
        lo_v = _k2f(lo_k)
        hi_v = _k2f(hi_k)

        width1 = jnp.where(hi_k == lo_k + one, 1 - done, 0)
        bound = jnp.where(width1 == 1, lo_v, bound)
        done = done | width1

        r = (logk - lhi) / (llo - lhi)
        p_interp = hi_v + (lo_v - hi_v) * r
        pk_i = _f2k(p_interp)
        pk_m = (lo_k >> 1) + (hi_k >> 1) + (lo_k & hi_k & one)
        pk = jnp.where(j < _INTERP_ITERS, pk_i, pk_m)
        pk = jnp.clip(pk, lo_k + one, hi_k - one)
        p_v = _k2f(pk)

        cnt = jnp.sum(jnp.where(x >= p_v, 1.0, 0.0), axis=1, keepdims=True)

        hit_k = jnp.where(cnt == kf, 1 - done, 0)
        bound = jnp.where(hit_k == 1, p_v, bound)
        done = done | hit_k

        lp = jnp.log(jnp.maximum(cnt, 0.5))
        live = done == 0
        take_lo = jnp.logical_and(live, cnt >= kf)
        take_hi = jnp.logical_and(live, cnt < kf)
        lo_k = jnp.where(take_lo, pk, lo_k)
        llo = jnp.where(take_lo, lp, llo)
        hi_k = jnp.where(take_hi, pk, hi_k)
        lhi = jnp.where(take_hi, lp, lhi)
        return j + one, lo_k, hi_k, llo, lhi, done, bound

    state0 = (
        jnp.int32(0),
        lo_k,
        hi_k,
        llo,
        lhi,
        jnp.zeros((nrows, 1), jnp.int32),
        jnp.zeros((nrows, 1), jnp.float32),
    )
    st = jax.lax.while_loop(cond, body, state0)
    _, lo_k, _, _, _, done, bound = st
    bound = jnp.where(done == 0, _k2f(lo_k), bound)

    o_ref[...] = jnp.where(x >= bound, x, jnp.float32(0.0))


def kernel(x):
    return pl.pallas_call(
        _topk_mask_kernel,
        grid=(_ROWS // _BLOCK_R,),
        in_specs=[pl.BlockSpec((_BLOCK_R, _COLS), lambda r: (r, 0))],
        out_specs=pl.BlockSpec((_BLOCK_R, _COLS), lambda r: (r, 0)),
        out_shape=jax.ShapeDtypeStruct((_ROWS, _COLS), jnp.float32),
    )(x)
